# baseline (speedup 1.0000x reference)
import functools

import jax
import jax.numpy as jnp
from jax import lax
from jax.experimental import pallas as pl
from jax.experimental.pallas import tpu as pltpu
from jax.experimental.pallas import tpu_sc as plsc

_N = 10000
_E = 160000
_F = 256
_CUT = 1.0

_NC = 2
_NS = 16
_NW = _NC * _NS
_CHUNK = 128
_EP = 163840
_CPW = _EP // (_NW * _CHUNK)

_NBLK = 400
_EBLK = 512

_FQ = 64
_NP = 10240
_RPT = _NP // _NS


def _sig(t):
    return 1.0 / (1.0 + jnp.exp(-t))


def _silu(t):
    return t * _sig(t)


def _mm(a, b):
    return lax.dot_general(a, b, (((1,), (0,)), ((), ())),
                           preferred_element_type=jnp.float32)


def _mlp2(xb, w1, b1, w2, b2):
    h = _silu(_mm(xb, w1) + b1)
    return _mm(h, w2) + b2


def _node_body(xb, wk1, bk1, wk2, bk2, wq1, bq1, wq2, bq2,
               wv1, bv1, wv2, bv2, ws11, bs11, ws12, bs12,
               ws21, bs21, ws22, bs22, ko, qo, vo, s1o, s2o):
    x = xb[...]
    ko[...] = _mlp2(x, wk1[...], bk1[...], wk2[...], bk2[...])
    qo[...] = _mlp2(x, wq1[...], bq1[...], wq2[...], bq2[...])
    v = _mlp2(x, wv1[...], bv1[...], wv2[...], bv2[...])
    vo[...] = v
    s1o[...] = _mlp2(v, ws11[...], bs11[...], ws12[...], bs12[...])
    s2o[...] = _mlp2(v, ws21[...], bs21[...], ws22[...], bs22[...])


def _node_mlps(x, params):
    n = x.shape[0]
    grid = (n // _NBLK,)
    xspec = pl.BlockSpec((_NBLK, _F), lambda i: (i, 0))
    wspec = pl.BlockSpec((_F, _F), lambda i: (0, 0))
    bspec = pl.BlockSpec((1, _F), lambda i: (0, 0))
    ospec = pl.BlockSpec((_NBLK, _F), lambda i: (i, 0))
    args = []
    for name in ('K', 'Q', 'V', 'S1', 'S2'):
        p = params[name]
        args += [p['W1'], p['b1'].reshape(1, _F), p['W2'], p['b2'].reshape(1, _F)]
    in_specs = [xspec] + [wspec, bspec, wspec, bspec] * 5
    out = jax.ShapeDtypeStruct((n, _F), jnp.float32)
    return pl.pallas_call(
        _node_body, grid=grid, in_specs=in_specs,
        out_specs=[ospec] * 5, out_shape=[out] * 5,
    )(x, *args)


_GROW = 64
_NBUF = 4
_CPT_FAST = 112
_CPT_SLOW = 160 - _CPT_FAST


def _gather_phase(table_hbm, out_hbm, idx_v, rows, chunk0, quarter,
                  gsems, wsems):
    def gath(t, b):
        pltpu.async_copy(table_hbm.at[idx_v.at[t]], rows.at[b], gsems[b])

    def wb(t, b):
        pltpu.async_copy(rows.at[b],
                         out_hbm.at[pl.ds((chunk0 + t) * _GROW, _GROW)],
                         wsems[b])

    def gwait(b):
        pltpu.make_async_copy(table_hbm.at[idx_v.at[0]], rows.at[b],
                              gsems[b]).wait()

    def wwait(b):
        pltpu.make_async_copy(rows.at[b], out_hbm.at[pl.ds(0, _GROW)],
                              wsems[b]).wait()

    for b in range(_NBUF):
        gath(b, b)

    def step(i, carry):
        t0 = _NBUF * i
        for b in range(_NBUF):
            gwait(b)
            wb(t0 + b, b)

        @pl.when(i < quarter - 1)
        def _():
            for b in range(_NBUF):
                wwait(b)
                gath(t0 + _NBUF + b, b)

        return carry

    lax.fori_loop(0, quarter, step, 0)
    for b in range(_NBUF):
        wwait(b)


def _gather_body(k_hbm, q_hbm, v_hbm, src_hbm, dst_hbm,
                 kg_hbm, qg_hbm, vg_hbm, sidx_v, didx_v, rows, isem,
                 g0, g1, g2, g3, w0, w1, w2, w3):
    cid = lax.axis_index("c")
    sid = lax.axis_index("s")
    cpt = _CPT_FAST - (_CPT_FAST - _CPT_SLOW) * cid
    chunk0 = cid * (_NS * _CPT_FAST) + sid * cpt
    quarter = cpt // _NBUF
    gsems = (g0, g1, g2, g3)
    wsems = (w0, w1, w2, w3)
    pltpu.async_copy(src_hbm.at[pl.ds(chunk0, _CPT_FAST)], sidx_v, isem)
    pltpu.async_copy(dst_hbm.at[pl.ds(chunk0, _CPT_FAST)], didx_v, isem)
    pltpu.make_async_copy(src_hbm.at[pl.ds(0, _CPT_FAST)], sidx_v, isem).wait()
    pltpu.make_async_copy(dst_hbm.at[pl.ds(0, _CPT_FAST)], didx_v, isem).wait()
    _gather_phase(k_hbm, kg_hbm, sidx_v, rows, chunk0, quarter, gsems, wsems)
    _gather_phase(v_hbm, vg_hbm, sidx_v, rows, chunk0, quarter, gsems, wsems)
    _gather_phase(q_hbm, qg_hbm, didx_v, rows, chunk0, quarter, gsems, wsems)


def _gather3(k, q, v, src2d, dst2d):
    mesh = plsc.VectorSubcoreMesh(core_axis_name="c", subcore_axis_name="s",
                                  num_cores=_NC, num_subcores=_NS)
    out = jax.ShapeDtypeStruct((_EP, _F), jnp.float32)
    npad = _NS * _CPT_FAST + (_NS - 1) * _CPT_SLOW + _CPT_FAST
    src_p = jnp.pad(src2d, ((0, npad - src2d.shape[0]), (0, 0)))
    dst_p = jnp.pad(dst2d, ((0, npad - dst2d.shape[0]), (0, 0)))
    fn = pl.kernel(
        _gather_body, out_type=[out, out, out], mesh=mesh,
        scratch_types=[
            pltpu.VMEM((_CPT_FAST, _GROW), jnp.int32),
            pltpu.VMEM((_CPT_FAST, _GROW), jnp.int32),
            pltpu.VMEM((_NBUF, _GROW, _F), jnp.float32),
            pltpu.SemaphoreType.DMA,
            pltpu.SemaphoreType.DMA,
            pltpu.SemaphoreType.DMA,
            pltpu.SemaphoreType.DMA,
            pltpu.SemaphoreType.DMA,
            pltpu.SemaphoreType.DMA,
            pltpu.SemaphoreType.DMA,
            pltpu.SemaphoreType.DMA,
            pltpu.SemaphoreType.DMA,
        ],
    )
    return fn(k, q, v, src_p, dst_p)


def _edge_body(ev, kg, qg, vg, cen, wk1, bk1, wk2, bk2,
               wv1, bv1, wv2, bv2, mo):
    i = pl.program_id(0)
    d = ev[0, 0]
    cut = jnp.where(d < _CUT, 0.5 * (jnp.cos(jnp.pi * d / _CUT) + 1.0), 0.0)
    gamma = jnp.float32(_F) / _CUT
    diff = d[:, None] - cen[...]
    bf = jnp.exp(-gamma * diff * diff)
    dk = _silu(_mlp2(bf, wk1[...], bk1[...], wk2[...], bk2[...]) * cut[:, None])
    dv = _silu(_mlp2(bf, wv1[...], bv1[...], wv2[...], bv2[...]) * cut[:, None])
    ke = kg[...] * dk
    wdot = jnp.sum(ke * qg[...], axis=-1)
    weight = _silu(wdot) * cut * (1.0 / jnp.sqrt(jnp.float32(_F)))
    h = vg[...] * dv * weight[:, None]
    eid = i * _EBLK + lax.broadcasted_iota(jnp.int32, (_EBLK, 1), 0)
    valid = eid < _E
    logh = jnp.where(valid, jnp.log(jnp.abs(h)), 0.0)
    sgn = jnp.where(valid & (h < 0), 1.0, 0.0)
    parts = []
    for b in range(4):
        parts.append(logh[:, b * _FQ:(b + 1) * _FQ])
        parts.append(sgn[:, b * _FQ:(b + 1) * _FQ])
    mo[...] = jnp.concatenate(parts, axis=1)


def _edge_stage(ev2d, kg, qg, vg, centers, params):
    grid = (_EP // _EBLK,)
    espec = pl.BlockSpec((1, 1, _EBLK), lambda i: (i, 0, 0))
    gspec = pl.BlockSpec((_EBLK, _F), lambda i: (i, 0))
    cspec = pl.BlockSpec((1, _F), lambda i: (0, 0))
    wspec = pl.BlockSpec((_F, _F), lambda i: (0, 0))
    bspec = pl.BlockSpec((1, _F), lambda i: (0, 0))
    args = []
    for name in ('dK', 'dV'):
        p = params[name]
        args += [p['W1'], p['b1'].reshape(1, _F), p['W2'], p['b2'].reshape(1, _F)]
    out = jax.ShapeDtypeStruct((_EP, 2 * _F), jnp.float32)
    ospec = pl.BlockSpec((_EBLK, 2 * _F), lambda i: (i, 0))
    return pl.pallas_call(
        _edge_body, grid=grid,
        in_specs=[espec, gspec, gspec, gspec, cspec] + [wspec, bspec] * 4,
        out_specs=ospec, out_shape=out,
    )(ev2d, kg, qg, vg, centers.reshape(1, _F), *args)


def _scatter_body(m_hbm, dst_hbm, zero_hbm, agg_hbm, idx_v, val2, acc,
                  isem, g0, g1):
    cid = lax.axis_index("c")
    sid = lax.axis_index("s")
    row0 = sid * _RPT
    cpt = _EP // _CHUNK // _NS
    half = cpt // 2
    b0 = val2.at[0]
    b1 = val2.at[1]

    pltpu.async_copy(dst_hbm.at[pl.ds(sid * cpt, cpt)], idx_v, isem)
    pltpu.make_async_copy(dst_hbm.at[pl.ds(0, cpt)], idx_v, isem).wait()

    for p in range(2):
        f0 = (cid * 2 + p) * 2 * _FQ
        pltpu.sync_copy(zero_hbm.at[pl.ds(row0, _RPT)], acc.at[pl.ds(row0, _RPT)])
        plsc.subcore_barrier()

        def load(t, rbuf, sem):
            base = (sid * cpt + t) * _CHUNK
            pltpu.async_copy(
                m_hbm.at[pl.ds(base, _CHUNK), pl.ds(f0, 2 * _FQ)], rbuf, sem)

        def lwait(rbuf, sem):
            pltpu.make_async_copy(
                m_hbm.at[pl.ds(0, _CHUNK), pl.ds(f0, 2 * _FQ)], rbuf,
                sem).wait()

        load(0, b0, g0)
        load(1, b1, g1)

        def step(i, carry):
            t0 = 2 * i
            lwait(b0, g0)
            pltpu.sync_copy(b0, acc.at[idx_v.at[t0]], add=True)
            lwait(b1, g1)
            pltpu.sync_copy(b1, acc.at[idx_v.at[t0 + 1]], add=True)

            @pl.when(i < half - 1)
            def _():
                load(t0 + 2, b0, g0)
                load(t0 + 3, b1, g1)

            return carry

        lax.fori_loop(0, half, step, 0)
        plsc.subcore_barrier()
        pltpu.sync_copy(acc.at[pl.ds(row0, _RPT)],
                        agg_hbm.at[pl.ds(row0, _RPT), pl.ds(f0, 2 * _FQ)])
        plsc.subcore_barrier()


def _scatter2(m, dst2d, zeros):
    mesh = plsc.VectorSubcoreMesh(core_axis_name="c", subcore_axis_name="s",
                                  num_cores=_NC, num_subcores=_NS)
    out = jax.ShapeDtypeStruct((_NP, 2 * _F), jnp.float32)
    cpt = _EP // _CHUNK // _NS
    fn = pl.kernel(
        _scatter_body, out_type=out, mesh=mesh,
        scratch_types=[
            pltpu.VMEM((cpt, _CHUNK), jnp.int32),
            pltpu.VMEM((2, _CHUNK, 2 * _FQ), jnp.float32),
            pltpu.VMEM_SHARED((_NP, 2 * _FQ), jnp.float32),
            pltpu.SemaphoreType.DMA,
            pltpu.SemaphoreType.DMA,
            pltpu.SemaphoreType.DMA,
        ],
    )
    return fn(m, dst2d, zeros)


def _out_body(mb, w1, b1, w2, b2, yo):
    m = mb[...]
    lparts, sparts = [], []
    for b in range(4):
        lparts.append(m[:, (2 * b) * _FQ:(2 * b + 1) * _FQ])
        sparts.append(m[:, (2 * b + 1) * _FQ:(2 * b + 2) * _FQ])
    lagg = jnp.concatenate(lparts, axis=1)
    nagg = jnp.concatenate(sparts, axis=1)
    odd = jnp.mod(nagg, 2.0)
    hagg = (1.0 - 2.0 * odd) * jnp.exp(lagg)
    yo[...] = _mlp2(hagg, w1[...], b1[...], w2[...], b2[...])


def _out_mlp(agg, params):
    grid = (_NP // 512,)
    mspec = pl.BlockSpec((512, 2 * _F), lambda i: (i, 0))
    spec = pl.BlockSpec((512, _F), lambda i: (i, 0))
    wspec = pl.BlockSpec((_F, _F), lambda i: (0, 0))
    bspec = pl.BlockSpec((1, _F), lambda i: (0, 0))
    p = params['IB']
    return pl.pallas_call(
        _out_body, grid=grid,
        in_specs=[mspec, wspec, bspec, wspec, bspec],
        out_specs=spec, out_shape=jax.ShapeDtypeStruct((_NP, _F), jnp.float32),
    )(agg, p['W1'], p['b1'].reshape(1, _F), p['W2'], p['b2'].reshape(1, _F))


def kernel(x, edge_index, e_var, params):
    src = edge_index[0].astype(jnp.int32)
    dst = edge_index[1].astype(jnp.int32)
    pad = _EP - _E
    src2d = jnp.pad(src, (0, pad)).reshape(_EP // _CHUNK, _CHUNK)
    dst2d = jnp.pad(dst, (0, pad)).reshape(_EP // _CHUNK, _CHUNK)
    src64 = jnp.pad(src, (0, pad)).reshape(_EP // _GROW, _GROW)
    dst64 = jnp.pad(dst, (0, pad)).reshape(_EP // _GROW, _GROW)
    ev2d = jnp.pad(e_var, (0, pad)).reshape(_EP // _EBLK, 1, _EBLK)
    centers = jnp.linspace(0.0, _CUT, _F, dtype=jnp.float32)
    zeros = jnp.zeros((_NP, 2 * _FQ), jnp.float32)

    k, q, v, s1, s2 = _node_mlps(x, params)
    kg, qg, vg = _gather3(k, q, v, src64, dst64)
    m = _edge_stage(ev2d, kg, qg, vg, centers, params)
    agg = _scatter2(m, dst2d, zeros)
    y = _out_mlp(agg, params)
    return (y[:_N], s1, s2)


# speedup vs baseline: 1.0193x; 1.0193x over previous
import functools

import jax
import jax.numpy as jnp
from jax import lax
from jax.experimental import pallas as pl
from jax.experimental.pallas import tpu as pltpu
from jax.experimental.pallas import tpu_sc as plsc

_N = 10000
_E = 160000
_F = 256
_CUT = 1.0

_NC = 2
_NS = 16
_NW = _NC * _NS
_CHUNK = 128
_EP = 163840
_CPW = _EP // (_NW * _CHUNK)

_NBLK = 400
_EBLK = 512

_FQ = 64
_NP = 10240
_RPT = _NP // _NS


def _sig(t):
    return 1.0 / (1.0 + jnp.exp(-t))


def _silu(t):
    return t * _sig(t)


def _mm(a, b):
    return lax.dot_general(a, b, (((1,), (0,)), ((), ())),
                           preferred_element_type=jnp.float32)


def _mlp2(xb, w1, b1, w2, b2):
    h = _silu(_mm(xb, w1) + b1)
    return _mm(h, w2) + b2


def _node_body(xb, wk1, bk1, wk2, bk2, wq1, bq1, wq2, bq2,
               wv1, bv1, wv2, bv2, ws11, bs11, ws12, bs12,
               ws21, bs21, ws22, bs22, ko, qo, vo, s1o, s2o):
    x = xb[...]
    ko[...] = _mlp2(x, wk1[...], bk1[...], wk2[...], bk2[...])
    qo[...] = _mlp2(x, wq1[...], bq1[...], wq2[...], bq2[...])
    v = _mlp2(x, wv1[...], bv1[...], wv2[...], bv2[...])
    vo[...] = v
    s1o[...] = _mlp2(v, ws11[...], bs11[...], ws12[...], bs12[...])
    s2o[...] = _mlp2(v, ws21[...], bs21[...], ws22[...], bs22[...])


def _node_mlps(x, params):
    n = x.shape[0]
    grid = (n // _NBLK,)
    xspec = pl.BlockSpec((_NBLK, _F), lambda i: (i, 0))
    wspec = pl.BlockSpec((_F, _F), lambda i: (0, 0))
    bspec = pl.BlockSpec((1, _F), lambda i: (0, 0))
    ospec = pl.BlockSpec((_NBLK, _F), lambda i: (i, 0))
    args = []
    for name in ('K', 'Q', 'V', 'S1', 'S2'):
        p = params[name]
        args += [p['W1'], p['b1'].reshape(1, _F), p['W2'], p['b2'].reshape(1, _F)]
    in_specs = [xspec] + [wspec, bspec, wspec, bspec] * 5
    out = jax.ShapeDtypeStruct((n, _F), jnp.float32)
    return pl.pallas_call(
        _node_body, grid=grid, in_specs=in_specs,
        out_specs=[ospec] * 5, out_shape=[out] * 5,
    )(x, *args)


_GROW = 128
_NBUF = 2
_CPT_FAST = 64
_CPT_SLOW = 80 - _CPT_FAST


def _gather_phase(table_hbm, out_hbm, idx_v, rows, chunk0, quarter,
                  gsems, wsems):
    def gath(t, b):
        pltpu.async_copy(table_hbm.at[idx_v.at[t]], rows.at[b], gsems[b])

    def wb(t, b):
        pltpu.async_copy(rows.at[b],
                         out_hbm.at[pl.ds((chunk0 + t) * _GROW, _GROW)],
                         wsems[b])

    def gwait(b):
        pltpu.make_async_copy(table_hbm.at[idx_v.at[0]], rows.at[b],
                              gsems[b]).wait()

    def wwait(b):
        pltpu.make_async_copy(rows.at[b], out_hbm.at[pl.ds(0, _GROW)],
                              wsems[b]).wait()

    for b in range(_NBUF):
        gath(b, b)

    def step(i, carry):
        t0 = _NBUF * i
        for b in range(_NBUF):
            gwait(b)
            wb(t0 + b, b)

        @pl.when(i < quarter - 1)
        def _():
            for b in range(_NBUF):
                wwait(b)
                gath(t0 + _NBUF + b, b)

        return carry

    lax.fori_loop(0, quarter, step, 0)
    for b in range(_NBUF):
        wwait(b)


def _gather_body(k_hbm, q_hbm, v_hbm, src_hbm, dst_hbm,
                 kg_hbm, qg_hbm, vg_hbm, sidx_v, didx_v, rows, isem,
                 g0, g1, g2, g3, w0, w1, w2, w3):
    cid = lax.axis_index("c")
    sid = lax.axis_index("s")
    cpt = _CPT_FAST - (_CPT_FAST - _CPT_SLOW) * cid
    chunk0 = cid * (_NS * _CPT_FAST) + sid * cpt
    quarter = cpt // _NBUF
    gsems = (g0, g1, g2, g3)
    wsems = (w0, w1, w2, w3)
    pltpu.async_copy(src_hbm.at[pl.ds(chunk0, _CPT_FAST)], sidx_v, isem)
    pltpu.async_copy(dst_hbm.at[pl.ds(chunk0, _CPT_FAST)], didx_v, isem)
    pltpu.make_async_copy(src_hbm.at[pl.ds(0, _CPT_FAST)], sidx_v, isem).wait()
    pltpu.make_async_copy(dst_hbm.at[pl.ds(0, _CPT_FAST)], didx_v, isem).wait()
    _gather_phase(k_hbm, kg_hbm, sidx_v, rows, chunk0, quarter, gsems, wsems)
    _gather_phase(v_hbm, vg_hbm, sidx_v, rows, chunk0, quarter, gsems, wsems)
    _gather_phase(q_hbm, qg_hbm, didx_v, rows, chunk0, quarter, gsems, wsems)


def _gather3(k, q, v, src2d, dst2d):
    mesh = plsc.VectorSubcoreMesh(core_axis_name="c", subcore_axis_name="s",
                                  num_cores=_NC, num_subcores=_NS)
    out = jax.ShapeDtypeStruct((_EP, _F), jnp.float32)
    npad = _NS * _CPT_FAST + (_NS - 1) * _CPT_SLOW + _CPT_FAST
    src_p = jnp.pad(src2d, ((0, npad - src2d.shape[0]), (0, 0)))
    dst_p = jnp.pad(dst2d, ((0, npad - dst2d.shape[0]), (0, 0)))
    fn = pl.kernel(
        _gather_body, out_type=[out, out, out], mesh=mesh,
        scratch_types=[
            pltpu.VMEM((_CPT_FAST, _GROW), jnp.int32),
            pltpu.VMEM((_CPT_FAST, _GROW), jnp.int32),
            pltpu.VMEM((_NBUF, _GROW, _F), jnp.float32),
            pltpu.SemaphoreType.DMA,
            pltpu.SemaphoreType.DMA,
            pltpu.SemaphoreType.DMA,
            pltpu.SemaphoreType.DMA,
            pltpu.SemaphoreType.DMA,
            pltpu.SemaphoreType.DMA,
            pltpu.SemaphoreType.DMA,
            pltpu.SemaphoreType.DMA,
            pltpu.SemaphoreType.DMA,
        ],
    )
    return fn(k, q, v, src_p, dst_p)


def _edge_body(ev, kg, qg, vg, cen, wk1, bk1, wk2, bk2,
               wv1, bv1, wv2, bv2, mo):
    i = pl.program_id(0)
    d = ev[0, 0]
    cut = jnp.where(d < _CUT, 0.5 * (jnp.cos(jnp.pi * d / _CUT) + 1.0), 0.0)
    gamma = jnp.float32(_F) / _CUT
    diff = d[:, None] - cen[...]
    bf = jnp.exp(-gamma * diff * diff)
    dk = _silu(_mlp2(bf, wk1[...], bk1[...], wk2[...], bk2[...]) * cut[:, None])
    dv = _silu(_mlp2(bf, wv1[...], bv1[...], wv2[...], bv2[...]) * cut[:, None])
    ke = kg[...] * dk
    wdot = jnp.sum(ke * qg[...], axis=-1)
    weight = _silu(wdot) * cut * (1.0 / jnp.sqrt(jnp.float32(_F)))
    h = vg[...] * dv * weight[:, None]
    eid = i * _EBLK + lax.broadcasted_iota(jnp.int32, (_EBLK, 1), 0)
    valid = eid < _E
    logh = jnp.where(valid, jnp.log(jnp.abs(h)), 0.0)
    sgn = jnp.where(valid & (h < 0), 1.0, 0.0)
    parts = []
    for b in range(4):
        parts.append(logh[:, b * _FQ:(b + 1) * _FQ])
        parts.append(sgn[:, b * _FQ:(b + 1) * _FQ])
    mo[...] = jnp.concatenate(parts, axis=1)


def _edge_stage(ev2d, kg, qg, vg, centers, params):
    grid = (_EP // _EBLK,)
    espec = pl.BlockSpec((1, 1, _EBLK), lambda i: (i, 0, 0))
    gspec = pl.BlockSpec((_EBLK, _F), lambda i: (i, 0))
    cspec = pl.BlockSpec((1, _F), lambda i: (0, 0))
    wspec = pl.BlockSpec((_F, _F), lambda i: (0, 0))
    bspec = pl.BlockSpec((1, _F), lambda i: (0, 0))
    args = []
    for name in ('dK', 'dV'):
        p = params[name]
        args += [p['W1'], p['b1'].reshape(1, _F), p['W2'], p['b2'].reshape(1, _F)]
    out = jax.ShapeDtypeStruct((_EP, 2 * _F), jnp.float32)
    ospec = pl.BlockSpec((_EBLK, 2 * _F), lambda i: (i, 0))
    return pl.pallas_call(
        _edge_body, grid=grid,
        in_specs=[espec, gspec, gspec, gspec, cspec] + [wspec, bspec] * 4,
        out_specs=ospec, out_shape=out,
    )(ev2d, kg, qg, vg, centers.reshape(1, _F), *args)


def _scatter_body(m_hbm, dst_hbm, zero_hbm, agg_hbm, idx_v, val2, acc,
                  isem, g0, g1):
    cid = lax.axis_index("c")
    sid = lax.axis_index("s")
    row0 = sid * _RPT
    cpt = _EP // _CHUNK // _NS
    half = cpt // 2
    b0 = val2.at[0]
    b1 = val2.at[1]

    pltpu.async_copy(dst_hbm.at[pl.ds(sid * cpt, cpt)], idx_v, isem)
    pltpu.make_async_copy(dst_hbm.at[pl.ds(0, cpt)], idx_v, isem).wait()

    for p in range(2):
        f0 = (cid * 2 + p) * 2 * _FQ
        pltpu.sync_copy(zero_hbm.at[pl.ds(row0, _RPT)], acc.at[pl.ds(row0, _RPT)])
        plsc.subcore_barrier()

        def load(t, rbuf, sem):
            base = (sid * cpt + t) * _CHUNK
            pltpu.async_copy(
                m_hbm.at[pl.ds(base, _CHUNK), pl.ds(f0, 2 * _FQ)], rbuf, sem)

        def lwait(rbuf, sem):
            pltpu.make_async_copy(
                m_hbm.at[pl.ds(0, _CHUNK), pl.ds(f0, 2 * _FQ)], rbuf,
                sem).wait()

        load(0, b0, g0)
        load(1, b1, g1)

        def step(i, carry):
            t0 = 2 * i
            lwait(b0, g0)
            pltpu.sync_copy(b0, acc.at[idx_v.at[t0]], add=True)
            lwait(b1, g1)
            pltpu.sync_copy(b1, acc.at[idx_v.at[t0 + 1]], add=True)

            @pl.when(i < half - 1)
            def _():
                load(t0 + 2, b0, g0)
                load(t0 + 3, b1, g1)

            return carry

        lax.fori_loop(0, half, step, 0)
        plsc.subcore_barrier()
        pltpu.sync_copy(acc.at[pl.ds(row0, _RPT)],
                        agg_hbm.at[pl.ds(row0, _RPT), pl.ds(f0, 2 * _FQ)])
        plsc.subcore_barrier()


def _scatter2(m, dst2d, zeros):
    mesh = plsc.VectorSubcoreMesh(core_axis_name="c", subcore_axis_name="s",
                                  num_cores=_NC, num_subcores=_NS)
    out = jax.ShapeDtypeStruct((_NP, 2 * _F), jnp.float32)
    cpt = _EP // _CHUNK // _NS
    fn = pl.kernel(
        _scatter_body, out_type=out, mesh=mesh,
        scratch_types=[
            pltpu.VMEM((cpt, _CHUNK), jnp.int32),
            pltpu.VMEM((2, _CHUNK, 2 * _FQ), jnp.float32),
            pltpu.VMEM_SHARED((_NP, 2 * _FQ), jnp.float32),
            pltpu.SemaphoreType.DMA,
            pltpu.SemaphoreType.DMA,
            pltpu.SemaphoreType.DMA,
        ],
    )
    return fn(m, dst2d, zeros)


def _out_body(mb, w1, b1, w2, b2, yo):
    m = mb[...]
    lparts, sparts = [], []
    for b in range(4):
        lparts.append(m[:, (2 * b) * _FQ:(2 * b + 1) * _FQ])
        sparts.append(m[:, (2 * b + 1) * _FQ:(2 * b + 2) * _FQ])
    lagg = jnp.concatenate(lparts, axis=1)
    nagg = jnp.concatenate(sparts, axis=1)
    odd = jnp.mod(nagg, 2.0)
    hagg = (1.0 - 2.0 * odd) * jnp.exp(lagg)
    yo[...] = _mlp2(hagg, w1[...], b1[...], w2[...], b2[...])


def _out_mlp(agg, params):
    grid = (_NP // 512,)
    mspec = pl.BlockSpec((512, 2 * _F), lambda i: (i, 0))
    spec = pl.BlockSpec((512, _F), lambda i: (i, 0))
    wspec = pl.BlockSpec((_F, _F), lambda i: (0, 0))
    bspec = pl.BlockSpec((1, _F), lambda i: (0, 0))
    p = params['IB']
    return pl.pallas_call(
        _out_body, grid=grid,
        in_specs=[mspec, wspec, bspec, wspec, bspec],
        out_specs=spec, out_shape=jax.ShapeDtypeStruct((_NP, _F), jnp.float32),
    )(agg, p['W1'], p['b1'].reshape(1, _F), p['W2'], p['b2'].reshape(1, _F))


def kernel(x, edge_index, e_var, params):
    src = edge_index[0].astype(jnp.int32)
    dst = edge_index[1].astype(jnp.int32)
    pad = _EP - _E
    src2d = jnp.pad(src, (0, pad)).reshape(_EP // _CHUNK, _CHUNK)
    dst2d = jnp.pad(dst, (0, pad)).reshape(_EP // _CHUNK, _CHUNK)
    src64 = jnp.pad(src, (0, pad)).reshape(_EP // _GROW, _GROW)
    dst64 = jnp.pad(dst, (0, pad)).reshape(_EP // _GROW, _GROW)
    ev2d = jnp.pad(e_var, (0, pad)).reshape(_EP // _EBLK, 1, _EBLK)
    centers = jnp.linspace(0.0, _CUT, _F, dtype=jnp.float32)
    zeros = jnp.zeros((_NP, 2 * _FQ), jnp.float32)

    k, q, v, s1, s2 = _node_mlps(x, params)
    kg, qg, vg = _gather3(k, q, v, src64, dst64)
    m = _edge_stage(ev2d, kg, qg, vg, centers, params)
    agg = _scatter2(m, dst2d, zeros)
    y = _out_mlp(agg, params)
    return (y[:_N], s1, s2)


# speedup vs baseline: 1.1652x; 1.1431x over previous
import functools

import jax
import jax.numpy as jnp
from jax import lax
from jax.experimental import pallas as pl
from jax.experimental.pallas import tpu as pltpu
from jax.experimental.pallas import tpu_sc as plsc

_N = 10000
_E = 160000
_F = 256
_CUT = 1.0

_NC = 2
_NS = 16
_NW = _NC * _NS
_CHUNK = 128
_EP = 163840
_CPW = _EP // (_NW * _CHUNK)

_NBLK = 400
_EBLK = 512

_FQ = 64
_NP = 10240
_RPT = _NP // _NS


def _sig(t):
    return 1.0 / (1.0 + jnp.exp(-t))


def _silu(t):
    return t * _sig(t)


def _mm(a, b):
    return lax.dot_general(a, b, (((1,), (0,)), ((), ())),
                           preferred_element_type=jnp.float32)


def _mlp2(xb, w1, b1, w2, b2):
    h = _silu(_mm(xb, w1) + b1)
    return _mm(h, w2) + b2


def _node_body(xb, wk1, bk1, wk2, bk2, wq1, bq1, wq2, bq2,
               wv1, bv1, wv2, bv2, ws11, bs11, ws12, bs12,
               ws21, bs21, ws22, bs22, ko, qo, vo, s1o, s2o):
    x = xb[...]
    ko[...] = _mlp2(x, wk1[...], bk1[...], wk2[...], bk2[...])
    qo[...] = _mlp2(x, wq1[...], bq1[...], wq2[...], bq2[...])
    v = _mlp2(x, wv1[...], bv1[...], wv2[...], bv2[...])
    vo[...] = v
    s1o[...] = _mlp2(v, ws11[...], bs11[...], ws12[...], bs12[...])
    s2o[...] = _mlp2(v, ws21[...], bs21[...], ws22[...], bs22[...])


def _node_mlps(x, params):
    n = x.shape[0]
    grid = (n // _NBLK,)
    xspec = pl.BlockSpec((_NBLK, _F), lambda i: (i, 0))
    wspec = pl.BlockSpec((_F, _F), lambda i: (0, 0))
    bspec = pl.BlockSpec((1, _F), lambda i: (0, 0))
    ospec = pl.BlockSpec((_NBLK, _F), lambda i: (i, 0))
    args = []
    for name in ('K', 'Q', 'V', 'S1', 'S2'):
        p = params[name]
        args += [p['W1'], p['b1'].reshape(1, _F), p['W2'], p['b2'].reshape(1, _F)]
    in_specs = [xspec] + [wspec, bspec, wspec, bspec] * 5
    out = jax.ShapeDtypeStruct((n, _F), jnp.float32)
    return pl.pallas_call(
        _node_body, grid=grid, in_specs=in_specs,
        out_specs=[ospec] * 5, out_shape=[out] * 5,
    )(x, *args)


_GROW = 128
_NBUF = 2
_CPT_FAST = 64
_CPT_SLOW = 80 - _CPT_FAST


def _gather_phase(table_hbm, out_hbm, idx_v, rows, chunk0, quarter,
                  gsems, wsems):
    def gath(t, b):
        pltpu.async_copy(table_hbm.at[idx_v.at[t]], rows.at[b], gsems[b])

    def wb(t, b):
        pltpu.async_copy(rows.at[b],
                         out_hbm.at[pl.ds((chunk0 + t) * _GROW, _GROW)],
                         wsems[b])

    def gwait(b):
        pltpu.make_async_copy(table_hbm.at[idx_v.at[0]], rows.at[b],
                              gsems[b]).wait()

    def wwait(b):
        pltpu.make_async_copy(rows.at[b], out_hbm.at[pl.ds(0, _GROW)],
                              wsems[b]).wait()

    for b in range(_NBUF):
        gath(b, b)

    def step(i, carry):
        t0 = _NBUF * i
        for b in range(_NBUF):
            gwait(b)
            wb(t0 + b, b)

        @pl.when(i < quarter - 1)
        def _():
            for b in range(_NBUF):
                wwait(b)
                gath(t0 + _NBUF + b, b)

        return carry

    lax.fori_loop(0, quarter, step, 0)
    for b in range(_NBUF):
        wwait(b)


def _gather_body(cpt_fast, cpt_slow, k_hbm, q_hbm, v_hbm, src_hbm, dst_hbm,
                 kg_hbm, qg_hbm, vg_hbm, sidx_v, didx_v, rows, isem,
                 g0, g1, g2, g3, w0, w1, w2, w3):
    cid = lax.axis_index("c")
    sid = lax.axis_index("s")
    cpt = cpt_fast - (cpt_fast - cpt_slow) * cid
    chunk0 = cid * (_NS * cpt_fast) + sid * cpt
    quarter = cpt // _NBUF
    gsems = (g0, g1, g2, g3)
    wsems = (w0, w1, w2, w3)
    pltpu.async_copy(src_hbm.at[pl.ds(chunk0, cpt_fast)], sidx_v, isem)
    pltpu.async_copy(dst_hbm.at[pl.ds(chunk0, cpt_fast)], didx_v, isem)
    pltpu.make_async_copy(src_hbm.at[pl.ds(0, cpt_fast)], sidx_v, isem).wait()
    pltpu.make_async_copy(dst_hbm.at[pl.ds(0, cpt_fast)], didx_v, isem).wait()
    _gather_phase(k_hbm, kg_hbm, sidx_v, rows, chunk0, quarter, gsems, wsems)
    _gather_phase(v_hbm, vg_hbm, sidx_v, rows, chunk0, quarter, gsems, wsems)
    _gather_phase(q_hbm, qg_hbm, didx_v, rows, chunk0, quarter, gsems, wsems)


def _gather3(k, q, v, src2d, dst2d):
    nchunks = src2d.shape[0]
    cpt_fast = (nchunks * _CPT_FAST // 80) // _NS
    cpt_slow = nchunks // _NS - cpt_fast
    mesh = plsc.VectorSubcoreMesh(core_axis_name="c", subcore_axis_name="s",
                                  num_cores=_NC, num_subcores=_NS)
    out = jax.ShapeDtypeStruct((nchunks * _GROW, _F), jnp.float32)
    npad = _NS * cpt_fast + (_NS - 1) * cpt_slow + cpt_fast
    src_p = jnp.pad(src2d, ((0, npad - nchunks), (0, 0)))
    dst_p = jnp.pad(dst2d, ((0, npad - nchunks), (0, 0)))
    fn = pl.kernel(
        functools.partial(_gather_body, cpt_fast, cpt_slow),
        out_type=[out, out, out], mesh=mesh,
        scratch_types=[
            pltpu.VMEM((cpt_fast, _GROW), jnp.int32),
            pltpu.VMEM((cpt_fast, _GROW), jnp.int32),
            pltpu.VMEM((_NBUF, _GROW, _F), jnp.float32),
            pltpu.SemaphoreType.DMA,
            pltpu.SemaphoreType.DMA,
            pltpu.SemaphoreType.DMA,
            pltpu.SemaphoreType.DMA,
            pltpu.SemaphoreType.DMA,
            pltpu.SemaphoreType.DMA,
            pltpu.SemaphoreType.DMA,
            pltpu.SemaphoreType.DMA,
            pltpu.SemaphoreType.DMA,
        ],
    )
    return fn(k, q, v, src_p, dst_p)


def _edge_body(e0, ev, kg, qg, vg, cen, wk1, bk1, wk2, bk2,
               wv1, bv1, wv2, bv2, mo):
    i = pl.program_id(0)
    d = ev[0, 0]
    cut = jnp.where(d < _CUT, 0.5 * (jnp.cos(jnp.pi * d / _CUT) + 1.0), 0.0)
    gamma = jnp.float32(_F) / _CUT
    diff = d[:, None] - cen[...]
    bf = jnp.exp(-gamma * diff * diff)
    dk = _silu(_mlp2(bf, wk1[...], bk1[...], wk2[...], bk2[...]) * cut[:, None])
    dv = _silu(_mlp2(bf, wv1[...], bv1[...], wv2[...], bv2[...]) * cut[:, None])
    ke = kg[...] * dk
    wdot = jnp.sum(ke * qg[...], axis=-1)
    weight = _silu(wdot) * cut * (1.0 / jnp.sqrt(jnp.float32(_F)))
    h = vg[...] * dv * weight[:, None]
    eid = e0 + i * _EBLK + lax.broadcasted_iota(jnp.int32, (_EBLK, 1), 0)
    valid = eid < _E
    logh = jnp.where(valid, jnp.log(jnp.abs(h)), 0.0)
    sgn = jnp.where(valid & (h < 0), 1.0, 0.0)
    parts = []
    for b in range(4):
        parts.append(logh[:, b * _FQ:(b + 1) * _FQ])
        parts.append(sgn[:, b * _FQ:(b + 1) * _FQ])
    mo[...] = jnp.concatenate(parts, axis=1)


def _edge_stage(ev2d, kg, qg, vg, centers, params, e0=0):
    ep = kg.shape[0]
    grid = (ep // _EBLK,)
    espec = pl.BlockSpec((1, 1, _EBLK), lambda i: (i, 0, 0))
    gspec = pl.BlockSpec((_EBLK, _F), lambda i: (i, 0))
    cspec = pl.BlockSpec((1, _F), lambda i: (0, 0))
    wspec = pl.BlockSpec((_F, _F), lambda i: (0, 0))
    bspec = pl.BlockSpec((1, _F), lambda i: (0, 0))
    args = []
    for name in ('dK', 'dV'):
        p = params[name]
        args += [p['W1'], p['b1'].reshape(1, _F), p['W2'], p['b2'].reshape(1, _F)]
    out = jax.ShapeDtypeStruct((ep, 2 * _F), jnp.float32)
    ospec = pl.BlockSpec((_EBLK, 2 * _F), lambda i: (i, 0))
    return pl.pallas_call(
        functools.partial(_edge_body, e0), grid=grid,
        in_specs=[espec, gspec, gspec, gspec, cspec] + [wspec, bspec] * 4,
        out_specs=ospec, out_shape=out,
    )(ev2d, kg, qg, vg, centers.reshape(1, _F), *args)


def _scatter_body(cpt, m_hbm, dst_hbm, zero_hbm, agg_hbm, idx_v, val2, acc,
                  isem, g0, g1):
    cid = lax.axis_index("c")
    sid = lax.axis_index("s")
    row0 = sid * _RPT
    half = cpt // 2
    b0 = val2.at[0]
    b1 = val2.at[1]

    pltpu.async_copy(dst_hbm.at[pl.ds(sid * cpt, cpt)], idx_v, isem)
    pltpu.make_async_copy(dst_hbm.at[pl.ds(0, cpt)], idx_v, isem).wait()

    for p in range(2):
        f0 = (cid * 2 + p) * 2 * _FQ
        pltpu.sync_copy(zero_hbm.at[pl.ds(row0, _RPT), pl.ds(f0, 2 * _FQ)],
                        acc.at[pl.ds(row0, _RPT)])
        plsc.subcore_barrier()

        def load(t, rbuf, sem):
            base = (sid * cpt + t) * _CHUNK
            pltpu.async_copy(
                m_hbm.at[pl.ds(base, _CHUNK), pl.ds(f0, 2 * _FQ)], rbuf, sem)

        def lwait(rbuf, sem):
            pltpu.make_async_copy(
                m_hbm.at[pl.ds(0, _CHUNK), pl.ds(f0, 2 * _FQ)], rbuf,
                sem).wait()

        load(0, b0, g0)
        load(1, b1, g1)

        def step(i, carry):
            t0 = 2 * i
            lwait(b0, g0)
            pltpu.sync_copy(b0, acc.at[idx_v.at[t0]], add=True)
            lwait(b1, g1)
            pltpu.sync_copy(b1, acc.at[idx_v.at[t0 + 1]], add=True)

            @pl.when(i < half - 1)
            def _():
                load(t0 + 2, b0, g0)
                load(t0 + 3, b1, g1)

            return carry

        lax.fori_loop(0, half, step, 0)
        plsc.subcore_barrier()
        pltpu.sync_copy(acc.at[pl.ds(row0, _RPT)],
                        agg_hbm.at[pl.ds(row0, _RPT), pl.ds(f0, 2 * _FQ)])
        plsc.subcore_barrier()


def _scatter2(m, dst2d, init):
    mesh = plsc.VectorSubcoreMesh(core_axis_name="c", subcore_axis_name="s",
                                  num_cores=_NC, num_subcores=_NS)
    out = jax.ShapeDtypeStruct((_NP, 2 * _F), jnp.float32)
    cpt = m.shape[0] // _CHUNK // _NS
    fn = pl.kernel(
        functools.partial(_scatter_body, cpt), out_type=out, mesh=mesh,
        scratch_types=[
            pltpu.VMEM((cpt, _CHUNK), jnp.int32),
            pltpu.VMEM((2, _CHUNK, 2 * _FQ), jnp.float32),
            pltpu.VMEM_SHARED((_NP, 2 * _FQ), jnp.float32),
            pltpu.SemaphoreType.DMA,
            pltpu.SemaphoreType.DMA,
            pltpu.SemaphoreType.DMA,
        ],
    )
    return fn(m, dst2d, init)


def _out_body(mb, w1, b1, w2, b2, yo):
    m = mb[...]
    lparts, sparts = [], []
    for b in range(4):
        lparts.append(m[:, (2 * b) * _FQ:(2 * b + 1) * _FQ])
        sparts.append(m[:, (2 * b + 1) * _FQ:(2 * b + 2) * _FQ])
    lagg = jnp.concatenate(lparts, axis=1)
    nagg = jnp.concatenate(sparts, axis=1)
    odd = jnp.mod(nagg, 2.0)
    hagg = (1.0 - 2.0 * odd) * jnp.exp(lagg)
    yo[...] = _mlp2(hagg, w1[...], b1[...], w2[...], b2[...])


def _out_mlp(agg, params):
    grid = (_NP // 512,)
    mspec = pl.BlockSpec((512, 2 * _F), lambda i: (i, 0))
    spec = pl.BlockSpec((512, _F), lambda i: (i, 0))
    wspec = pl.BlockSpec((_F, _F), lambda i: (0, 0))
    bspec = pl.BlockSpec((1, _F), lambda i: (0, 0))
    p = params['IB']
    return pl.pallas_call(
        _out_body, grid=grid,
        in_specs=[mspec, wspec, bspec, wspec, bspec],
        out_specs=spec, out_shape=jax.ShapeDtypeStruct((_NP, _F), jnp.float32),
    )(agg, p['W1'], p['b1'].reshape(1, _F), p['W2'], p['b2'].reshape(1, _F))


def kernel(x, edge_index, e_var, params):
    src = edge_index[0].astype(jnp.int32)
    dst = edge_index[1].astype(jnp.int32)
    pad = _EP - _E
    srcp = jnp.pad(src, (0, pad))
    dstp = jnp.pad(dst, (0, pad))
    evp = jnp.pad(e_var, (0, pad))
    centers = jnp.linspace(0.0, _CUT, _F, dtype=jnp.float32)

    k, q, v, s1, s2 = _node_mlps(x, params)

    H = 2
    eph = _EP // H
    agg = jnp.zeros((_NP, 2 * _F), jnp.float32)
    gathered = []
    for h in range(H):
        s_h = lax.dynamic_slice_in_dim(srcp, h * eph, eph)
        d_h = lax.dynamic_slice_in_dim(dstp, h * eph, eph)
        gathered.append(
            _gather3(k, q, v, s_h.reshape(eph // _GROW, _GROW),
                     d_h.reshape(eph // _GROW, _GROW)))
    for h in range(H):
        kg, qg, vg = gathered[h]
        ev_h = lax.dynamic_slice_in_dim(evp, h * eph, eph)
        m = _edge_stage(ev_h.reshape(eph // _EBLK, 1, _EBLK), kg, qg, vg,
                        centers, params, e0=h * eph)
        d_h = lax.dynamic_slice_in_dim(dstp, h * eph, eph)
        agg = _scatter2(m, d_h.reshape(eph // _CHUNK, _CHUNK), agg)
    y = _out_mlp(agg, params)
    return (y[:_N], s1, s2)


# speedup vs baseline: 1.6638x; 1.4279x over previous
import functools

import jax
import jax.numpy as jnp
from jax import lax
from jax.experimental import pallas as pl
from jax.experimental.pallas import tpu as pltpu
from jax.experimental.pallas import tpu_sc as plsc

_N = 10000
_E = 160000
_F = 256
_CUT = 1.0

_NC = 2
_NS = 16
_NW = _NC * _NS
_CHUNK = 128
_EP = 163840
_CPW = _EP // (_NW * _CHUNK)

_NBLK = 512
_EBLK = 512

_FQ = 64
_NP = 10240
_RPT = _NP // _NS


def _sig(t):
    return 1.0 / (1.0 + jnp.exp(-t))


def _silu(t):
    return t * _sig(t)


def _mm(a, b):
    return lax.dot_general(a, b, (((1,), (0,)), ((), ())),
                           preferred_element_type=jnp.float32)


def _mlp2(xb, w1, b1, w2, b2):
    h = _silu(_mm(xb, w1) + b1)
    return _mm(h, w2) + b2


def _node_body(xb, wk1, bk1, wk2, bk2, wq1, bq1, wq2, bq2,
               wv1, bv1, wv2, bv2, ws11, bs11, ws12, bs12,
               ws21, bs21, ws22, bs22, ko, qo, vo, s1o, s2o):
    x = xb[...]
    ko[...] = _mlp2(x, wk1[...], bk1[...], wk2[...], bk2[...])
    qo[...] = _mlp2(x, wq1[...], bq1[...], wq2[...], bq2[...])
    v = _mlp2(x, wv1[...], bv1[...], wv2[...], bv2[...])
    vo[...] = v
    s1o[...] = _mlp2(v, ws11[...], bs11[...], ws12[...], bs12[...])
    s2o[...] = _mlp2(v, ws21[...], bs21[...], ws22[...], bs22[...])


def _node_mlps(x, params):
    n = x.shape[0]
    grid = (n // _NBLK,)
    xspec = pl.BlockSpec((_NBLK, _F), lambda i: (i, 0))
    wspec = pl.BlockSpec((_F, _F), lambda i: (0, 0))
    bspec = pl.BlockSpec((1, _F), lambda i: (0, 0))
    ospec = pl.BlockSpec((_NBLK, _F), lambda i: (i, 0))
    args = []
    for name in ('K', 'Q', 'V', 'S1', 'S2'):
        p = params[name]
        args += [p['W1'], p['b1'].reshape(1, _F), p['W2'], p['b2'].reshape(1, _F)]
    in_specs = [xspec] + [wspec, bspec, wspec, bspec] * 5
    out = jax.ShapeDtypeStruct((n, _F), jnp.float32)
    return pl.pallas_call(
        _node_body, grid=grid, in_specs=in_specs,
        out_specs=[ospec] * 5, out_shape=[out] * 5,
    )(x, *args)


_GROW = 128
_NBUF = 2
_CPT_FAST = 64
_CPT_SLOW = 80 - _CPT_FAST


_FH = 128


def _gather_phase(share, out_hbm, idx_v, rows, chunk0, quarter, c0,
                  gsems, wsems):
    def gath(t, b):
        pltpu.async_copy(share.at[idx_v.at[t]], rows.at[b], gsems[b])

    def wb(t, b):
        pltpu.async_copy(rows.at[b],
                         out_hbm.at[pl.ds((chunk0 + t) * _GROW, _GROW),
                                    pl.ds(c0, _FH)],
                         wsems[b])

    def gwait(b):
        pltpu.make_async_copy(share.at[idx_v.at[0]], rows.at[b],
                              gsems[b]).wait()

    def wwait(b):
        pltpu.make_async_copy(rows.at[b],
                              out_hbm.at[pl.ds(0, _GROW), pl.ds(c0, _FH)],
                              wsems[b]).wait()

    for b in range(_NBUF):
        gath(b, b)

    def step(i, carry):
        t0 = _NBUF * i
        for b in range(_NBUF):
            gwait(b)
            wb(t0 + b, b)

        @pl.when(i < quarter - 1)
        def _():
            for b in range(_NBUF):
                wwait(b)
                gath(t0 + _NBUF + b, b)

        return carry

    lax.fori_loop(0, quarter, step, 0)
    for b in range(_NBUF):
        wwait(b)


def _gather_body(cpt_fast, cpt_slow, k_hbm, q_hbm, v_hbm, src_hbm, dst_hbm,
                 kg_hbm, qg_hbm, vg_hbm, sidx_v, didx_v, rows, share, isem,
                 g0, g1, g2, g3, w0, w1, w2, w3):
    cid = lax.axis_index("c")
    sid = lax.axis_index("s")
    cpt = cpt_fast - (cpt_fast - cpt_slow) * cid
    chunk0 = cid * (_NS * cpt_fast) + sid * cpt
    quarter = cpt // _NBUF
    row0 = sid * (_NP // _NS)
    gsems = (g0, g1, g2, g3)
    wsems = (w0, w1, w2, w3)
    pltpu.async_copy(src_hbm.at[pl.ds(chunk0, cpt_fast)], sidx_v, isem)
    pltpu.async_copy(dst_hbm.at[pl.ds(chunk0, cpt_fast)], didx_v, isem)
    pltpu.make_async_copy(src_hbm.at[pl.ds(0, cpt_fast)], sidx_v, isem).wait()
    pltpu.make_async_copy(dst_hbm.at[pl.ds(0, cpt_fast)], didx_v, isem).wait()
    for table, idx_ref, out in ((k_hbm, sidx_v, kg_hbm),
                                (v_hbm, sidx_v, vg_hbm),
                                (q_hbm, didx_v, qg_hbm)):
        for c in range(_F // _FH):
            pltpu.sync_copy(
                table.at[pl.ds(row0, _NP // _NS), pl.ds(c * _FH, _FH)],
                share.at[pl.ds(row0, _NP // _NS)])
            plsc.subcore_barrier()
            _gather_phase(share, out, idx_ref, rows, chunk0, quarter,
                          c * _FH, gsems, wsems)
            plsc.subcore_barrier()


def _gather3(k, q, v, src2d, dst2d):
    nchunks = src2d.shape[0]
    cpt_fast = (nchunks * _CPT_FAST // 80) // _NS
    cpt_slow = nchunks // _NS - cpt_fast
    mesh = plsc.VectorSubcoreMesh(core_axis_name="c", subcore_axis_name="s",
                                  num_cores=_NC, num_subcores=_NS)
    out = jax.ShapeDtypeStruct((nchunks * _GROW, _F), jnp.float32)
    npad = _NS * cpt_fast + (_NS - 1) * cpt_slow + cpt_fast
    src_p = jnp.pad(src2d, ((0, npad - nchunks), (0, 0)))
    dst_p = jnp.pad(dst2d, ((0, npad - nchunks), (0, 0)))
    fn = pl.kernel(
        functools.partial(_gather_body, cpt_fast, cpt_slow),
        out_type=[out, out, out], mesh=mesh,
        scratch_types=[
            pltpu.VMEM((cpt_fast, _GROW), jnp.int32),
            pltpu.VMEM((cpt_fast, _GROW), jnp.int32),
            pltpu.VMEM((_NBUF, _GROW, _FH), jnp.float32),
            pltpu.VMEM_SHARED((_NP, _FH), jnp.float32),
            pltpu.SemaphoreType.DMA,
            pltpu.SemaphoreType.DMA,
            pltpu.SemaphoreType.DMA,
            pltpu.SemaphoreType.DMA,
            pltpu.SemaphoreType.DMA,
            pltpu.SemaphoreType.DMA,
            pltpu.SemaphoreType.DMA,
            pltpu.SemaphoreType.DMA,
            pltpu.SemaphoreType.DMA,
        ],
    )
    return fn(k, q, v, src_p, dst_p)


def _edge_body(e0, ev, kg, qg, vg, cen, wk1, bk1, wk2, bk2,
               wv1, bv1, wv2, bv2, mo):
    i = pl.program_id(0)
    d = ev[0, 0]
    cut = jnp.where(d < _CUT, 0.5 * (jnp.cos(jnp.pi * d / _CUT) + 1.0), 0.0)
    gamma = jnp.float32(_F) / _CUT
    diff = d[:, None] - cen[...]
    bf = jnp.exp(-gamma * diff * diff)
    dk = _silu(_mlp2(bf, wk1[...], bk1[...], wk2[...], bk2[...]) * cut[:, None])
    dv = _silu(_mlp2(bf, wv1[...], bv1[...], wv2[...], bv2[...]) * cut[:, None])
    ke = kg[...] * dk
    wdot = jnp.sum(ke * qg[...], axis=-1)
    weight = _silu(wdot) * cut * (1.0 / jnp.sqrt(jnp.float32(_F)))
    h = vg[...] * dv * weight[:, None]
    eid = e0 + i * _EBLK + lax.broadcasted_iota(jnp.int32, (_EBLK, 1), 0)
    valid = eid < _E
    logh = jnp.where(valid, jnp.log(jnp.abs(h)), 0.0)
    sgn = jnp.where(valid & (h < 0), 1.0, 0.0)
    parts = []
    for b in range(4):
        parts.append(logh[:, b * _FQ:(b + 1) * _FQ])
        parts.append(sgn[:, b * _FQ:(b + 1) * _FQ])
    mo[...] = jnp.concatenate(parts, axis=1)


def _edge_stage(ev2d, kg, qg, vg, centers, params, e0=0):
    ep = kg.shape[0]
    grid = (ep // _EBLK,)
    espec = pl.BlockSpec((1, 1, _EBLK), lambda i: (i, 0, 0))
    gspec = pl.BlockSpec((_EBLK, _F), lambda i: (i, 0))
    cspec = pl.BlockSpec((1, _F), lambda i: (0, 0))
    wspec = pl.BlockSpec((_F, _F), lambda i: (0, 0))
    bspec = pl.BlockSpec((1, _F), lambda i: (0, 0))
    args = []
    for name in ('dK', 'dV'):
        p = params[name]
        args += [p['W1'], p['b1'].reshape(1, _F), p['W2'], p['b2'].reshape(1, _F)]
    out = jax.ShapeDtypeStruct((ep, 2 * _F), jnp.float32)
    ospec = pl.BlockSpec((_EBLK, 2 * _F), lambda i: (i, 0))
    return pl.pallas_call(
        functools.partial(_edge_body, e0), grid=grid,
        in_specs=[espec, gspec, gspec, gspec, cspec] + [wspec, bspec] * 4,
        out_specs=ospec, out_shape=out,
    )(ev2d, kg, qg, vg, centers.reshape(1, _F), *args)


def _scatter_body(cpt, m_hbm, dst_hbm, zero_hbm, agg_hbm, idx_v, val2, acc,
                  isem, g0, g1):
    cid = lax.axis_index("c")
    sid = lax.axis_index("s")
    row0 = sid * _RPT
    half = cpt // 2
    b0 = val2.at[0]
    b1 = val2.at[1]

    pltpu.async_copy(dst_hbm.at[pl.ds(sid * cpt, cpt)], idx_v, isem)
    pltpu.make_async_copy(dst_hbm.at[pl.ds(0, cpt)], idx_v, isem).wait()

    for p in range(2):
        f0 = (cid * 2 + p) * 2 * _FQ
        pltpu.sync_copy(zero_hbm.at[pl.ds(row0, _RPT), pl.ds(f0, 2 * _FQ)],
                        acc.at[pl.ds(row0, _RPT)])
        plsc.subcore_barrier()

        def load(t, rbuf, sem):
            base = (sid * cpt + t) * _CHUNK
            pltpu.async_copy(
                m_hbm.at[pl.ds(base, _CHUNK), pl.ds(f0, 2 * _FQ)], rbuf, sem)

        def lwait(rbuf, sem):
            pltpu.make_async_copy(
                m_hbm.at[pl.ds(0, _CHUNK), pl.ds(f0, 2 * _FQ)], rbuf,
                sem).wait()

        load(0, b0, g0)
        load(1, b1, g1)

        def step(i, carry):
            t0 = 2 * i
            lwait(b0, g0)
            pltpu.sync_copy(b0, acc.at[idx_v.at[t0]], add=True)
            lwait(b1, g1)
            pltpu.sync_copy(b1, acc.at[idx_v.at[t0 + 1]], add=True)

            @pl.when(i < half - 1)
            def _():
                load(t0 + 2, b0, g0)
                load(t0 + 3, b1, g1)

            return carry

        lax.fori_loop(0, half, step, 0)
        plsc.subcore_barrier()
        pltpu.sync_copy(acc.at[pl.ds(row0, _RPT)],
                        agg_hbm.at[pl.ds(row0, _RPT), pl.ds(f0, 2 * _FQ)])
        plsc.subcore_barrier()


def _scatter2(m, dst2d, init):
    mesh = plsc.VectorSubcoreMesh(core_axis_name="c", subcore_axis_name="s",
                                  num_cores=_NC, num_subcores=_NS)
    out = jax.ShapeDtypeStruct((_NP, 2 * _F), jnp.float32)
    cpt = m.shape[0] // _CHUNK // _NS
    fn = pl.kernel(
        functools.partial(_scatter_body, cpt), out_type=out, mesh=mesh,
        scratch_types=[
            pltpu.VMEM((cpt, _CHUNK), jnp.int32),
            pltpu.VMEM((2, _CHUNK, 2 * _FQ), jnp.float32),
            pltpu.VMEM_SHARED((_NP, 2 * _FQ), jnp.float32),
            pltpu.SemaphoreType.DMA,
            pltpu.SemaphoreType.DMA,
            pltpu.SemaphoreType.DMA,
        ],
    )
    return fn(m, dst2d, init)


def _out_body(mb, w1, b1, w2, b2, yo):
    m = mb[...]
    lparts, sparts = [], []
    for b in range(4):
        lparts.append(m[:, (2 * b) * _FQ:(2 * b + 1) * _FQ])
        sparts.append(m[:, (2 * b + 1) * _FQ:(2 * b + 2) * _FQ])
    lagg = jnp.concatenate(lparts, axis=1)
    nagg = jnp.concatenate(sparts, axis=1)
    odd = jnp.mod(nagg, 2.0)
    hagg = (1.0 - 2.0 * odd) * jnp.exp(lagg)
    yo[...] = _mlp2(hagg, w1[...], b1[...], w2[...], b2[...])


def _out_mlp(agg, params):
    grid = (_NP // 512,)
    mspec = pl.BlockSpec((512, 2 * _F), lambda i: (i, 0))
    spec = pl.BlockSpec((512, _F), lambda i: (i, 0))
    wspec = pl.BlockSpec((_F, _F), lambda i: (0, 0))
    bspec = pl.BlockSpec((1, _F), lambda i: (0, 0))
    p = params['IB']
    return pl.pallas_call(
        _out_body, grid=grid,
        in_specs=[mspec, wspec, bspec, wspec, bspec],
        out_specs=spec, out_shape=jax.ShapeDtypeStruct((_NP, _F), jnp.float32),
    )(agg, p['W1'], p['b1'].reshape(1, _F), p['W2'], p['b2'].reshape(1, _F))


def kernel(x, edge_index, e_var, params):
    src = edge_index[0].astype(jnp.int32)
    dst = edge_index[1].astype(jnp.int32)
    pad = _EP - _E
    srcp = jnp.pad(src, (0, pad))
    dstp = jnp.pad(dst, (0, pad))
    evp = jnp.pad(e_var, (0, pad))
    centers = jnp.linspace(0.0, _CUT, _F, dtype=jnp.float32)

    xp = jnp.pad(x, ((0, _NP - _N), (0, 0)))
    k, q, v, s1, s2 = _node_mlps(xp, params)
    s1 = s1[:_N]
    s2 = s2[:_N]

    H = 2
    eph = _EP // H
    agg = jnp.zeros((_NP, 2 * _F), jnp.float32)
    gathered = []
    for h in range(H):
        s_h = lax.dynamic_slice_in_dim(srcp, h * eph, eph)
        d_h = lax.dynamic_slice_in_dim(dstp, h * eph, eph)
        gathered.append(
            _gather3(k, q, v, s_h.reshape(eph // _GROW, _GROW),
                     d_h.reshape(eph // _GROW, _GROW)))
    for h in range(H):
        kg, qg, vg = gathered[h]
        ev_h = lax.dynamic_slice_in_dim(evp, h * eph, eph)
        m = _edge_stage(ev_h.reshape(eph // _EBLK, 1, _EBLK), kg, qg, vg,
                        centers, params, e0=h * eph)
        d_h = lax.dynamic_slice_in_dim(dstp, h * eph, eph)
        agg = _scatter2(m, d_h.reshape(eph // _CHUNK, _CHUNK), agg)
    y = _out_mlp(agg, params)
    return (y[:_N], s1, s2)
